# baseline (speedup 1.0000x reference)
import functools

import jax
import jax.numpy as jnp
from jax import lax
from jax.experimental import pallas as pl
from jax.experimental.pallas import tpu as pltpu
from jax.experimental.pallas import tpu_sc as plsc

N = 10000
NP = 10240
E = 320000
CH = 128
NWORK = 32
CPW = 80
NCH = NWORK * CPW
EP = NCH * CH
CPW0 = 80
CPW1 = 2 * CPW - CPW0
CPWMAX = max(CPW0, CPW1)
NCH_PAD = 16 * CPW0 + 16 * CPW1 + CPWMAX + 8
SUP = 8
NSUP = CPWMAX // SUP
RPT = NP // 16
BLK = 1024
GRID = NP // BLK

_SC_PARAMS = pltpu.CompilerParams(use_tc_tiling_on_sc=False)


def _mesh():
  return plsc.VectorSubcoreMesh(core_axis_name="c", subcore_axis_name="s")


_SC_CACHE = {}


def _make_sc_segsum(D):
  if D in _SC_CACHE:
    return _SC_CACHE[D]

  @functools.partial(
      pl.kernel,
      mesh=_mesh(),
      compiler_params=_SC_PARAMS,
      out_type=jax.ShapeDtypeStruct((2, NP, D), jnp.float32),
      scratch_types=[
          pltpu.VMEM((CPWMAX, CH), jnp.int32),
          pltpu.VMEM((CPWMAX, CH), jnp.int32),
          pltpu.VMEM((CH, D), jnp.float32),
          pltpu.VMEM_SHARED((NP, D), jnp.float32),
          pltpu.SemaphoreType.DMA,
      ],
  )
  def seg(table, srcg, dstg, out, src_v, dst_v, rows_v, acc, sem):
    cid = lax.axis_index("c")
    sid = lax.axis_index("s")
    wbase = jnp.where(cid == 0, sid * CPW0, 16 * CPW0 + sid * CPW1)
    cnt = jnp.where(cid == 0, CPW0, CPW1)

    with jax.named_scope("zero"):
      def zrow(r, _):
        def zcol(c, _):
          rows_v[r, pl.ds(c * 16, 16)] = jnp.zeros((16,), jnp.float32)
          return 0
        return lax.fori_loop(0, D // 16, zcol, 0)
      lax.fori_loop(0, CH, zrow, 0)
      for k in range(RPT // CH):
        pltpu.sync_copy(rows_v, acc.at[pl.ds(sid * RPT + k * CH, CH)])
      plsc.subcore_barrier()

    with jax.named_scope("stage"):
      pltpu.sync_copy(srcg.at[pl.ds(wbase, CPWMAX)], src_v)
      pltpu.sync_copy(dstg.at[pl.ds(wbase, CPWMAX)], dst_v)

    with jax.named_scope("edges"):
      def body(j, _):
        pltpu.async_copy(table.at[src_v.at[j]], rows_v, sem).wait()
        pltpu.sync_copy(rows_v, acc.at[dst_v.at[j]], add=True)
        return 0
      lax.fori_loop(0, cnt, body, 0)
      plsc.subcore_barrier()

    with jax.named_scope("wb"):
      pltpu.sync_copy(acc.at[pl.ds(sid * RPT, RPT)],
                      out.at[cid, pl.ds(sid * RPT, RPT)])

  _SC_CACHE[D] = seg
  return seg


def _sc_segsum(table, srcg, dstg):
  return _make_sc_segsum(table.shape[1])(table, srcg, dstg)


PERM = sum([[32 * g + 2 * k for k in range(16)]
            + [32 * g + 2 * k + 1 for k in range(16)] for g in range(4)], [])


def _make_sc_segsum_bf():
  if "bf" in _SC_CACHE:
    return _SC_CACHE["bf"]

  @functools.partial(
      pl.kernel,
      mesh=_mesh(),
      compiler_params=_SC_PARAMS,
      out_type=jax.ShapeDtypeStruct((2, NP, 128), jnp.float32),
      scratch_types=[
          pltpu.VMEM((CPW + 1, CH), jnp.int32),
          pltpu.VMEM((SUP, CH), jnp.int32),
          pltpu.VMEM((CH, 64), jnp.int32),
          pltpu.VMEM((CH, 64), jnp.int32),
          pltpu.VMEM((CH, 128), jnp.float32),
          pltpu.VMEM_SHARED((NP, 128), jnp.float32),
          pltpu.SemaphoreType.DMA,
          pltpu.SemaphoreType.DMA,
      ],
  )
  def segbf(table, srcg, dstg, out, src_v, dst_v, bf0, bf1, rows_f, acc,
            g0, g1):
    cid = lax.axis_index("c")
    sid = lax.axis_index("s")
    wid = sid * 2 + cid
    wbase = wid * CPW
    bf = [bf0, bf1]
    gsem = [g0, g1]

    def issue_g(j, b):
      pltpu.async_copy(table.at[src_v.at[j]], bf[b], gsem[b])

    def wait_g(j, b):
      pltpu.make_async_copy(table.at[src_v.at[j]], bf[b], gsem[b]).wait()

    with jax.named_scope("zero"):
      def zrow(r, _):
        def zcol(c, _):
          rows_f[r, pl.ds(c * 16, 16)] = jnp.zeros((16,), jnp.float32)
          return 0
        return lax.fori_loop(0, 8, zcol, 0)
      lax.fori_loop(0, CH, zrow, 0)
      for k in range(RPT // CH):
        pltpu.sync_copy(rows_f, acc.at[pl.ds(sid * RPT + k * CH, CH)])
      for c in range(CH // 16):
        src_v[CPW, pl.ds(c * 16, 16)] = jnp.zeros((16,), jnp.int32)
      plsc.subcore_barrier()

    with jax.named_scope("stage"):
      pltpu.sync_copy(srcg.at[pl.ds(wbase, CPW)], src_v.at[pl.ds(0, CPW)])

    with jax.named_scope("edges"):
      issue_g(0, 0)

      def super_body(t, _):
        pltpu.sync_copy(dstg.at[pl.ds(wbase + t * SUP, SUP)], dst_v)
        for k in range(SUP):
          j = t * SUP + k
          b = k % 2
          wait_g(j, b)
          issue_g(j + 1, 1 - b)

          def crow(r4, _):
            for rr in range(4):
              r = r4 * 4 + rr
              for g in range(4):
                v = bf[b][r, pl.ds(g * 16, 16)]
                lo = lax.bitcast_convert_type(v << 16, jnp.float32)
                hi = lax.bitcast_convert_type(v & jnp.int32(-65536),
                                              jnp.float32)
                rows_f[r, pl.ds(g * 32, 16)] = lo
                rows_f[r, pl.ds(g * 32 + 16, 16)] = hi
            return 0
          lax.fori_loop(0, CH // 4, crow, 0)

          pltpu.sync_copy(rows_f, acc.at[dst_v.at[k]], add=True)
        return 0
      lax.fori_loop(0, NSUP, super_body, 0)

      wait_g(CPW, 0)
      plsc.subcore_barrier()

    with jax.named_scope("wb"):
      pltpu.sync_copy(acc.at[pl.ds(sid * RPT, RPT)],
                      out.at[cid, pl.ds(sid * RPT, RPT)])

  _SC_CACHE["bf"] = segbf
  return segbf


def _sc_segsum_bf(table_i32, srcg, dstg):
  return _make_sc_segsum_bf()(table_i32, srcg, dstg)


def _pack_bf(a):
  return lax.bitcast_convert_type(
      a.astype(jnp.bfloat16).reshape(NP, 64, 2), jnp.int32)


def _make_sc_deg():
  if "deg" in _SC_CACHE:
    return _SC_CACHE["deg"]

  @functools.partial(
      pl.kernel,
      mesh=_mesh(),
      compiler_params=_SC_PARAMS,
      out_type=jax.ShapeDtypeStruct((2, NP, 16), jnp.float32),
      scratch_types=[
          pltpu.VMEM((CPWMAX, CH), jnp.int32),
          pltpu.VMEM((CH, 16), jnp.float32),
          pltpu.VMEM_SHARED((NP, 16), jnp.float32),
          pltpu.SemaphoreType.DMA,
          pltpu.SemaphoreType.DMA,
      ],
  )
  def degk(dstg, out, dst_v, ones, acc, s0, s1):
    cid = lax.axis_index("c")
    sid = lax.axis_index("s")
    wbase = jnp.where(cid == 0, sid * CPW0, 16 * CPW0 + sid * CPW1)
    cnt = jnp.where(cid == 0, CPW0, CPW1)

    def fill(val):
      def frow(r, _):
        ones[r, pl.ds(0, 16)] = jnp.full((16,), val, jnp.float32)
        return 0
      lax.fori_loop(0, CH, frow, 0)

    fill(0.0)
    for k in range(RPT // CH):
      pltpu.sync_copy(ones, acc.at[pl.ds(sid * RPT + k * CH, CH)])
    plsc.subcore_barrier()
    fill(1.0)

    pltpu.sync_copy(dstg.at[pl.ds(wbase, CPWMAX)], dst_v)

    def body(t, _):
      pltpu.async_copy(ones, acc.at[dst_v.at[2 * t]], s0, add=True)
      pltpu.async_copy(ones, acc.at[dst_v.at[2 * t + 1]], s1, add=True)
      pltpu.make_async_copy(ones, acc.at[dst_v.at[2 * t]], s0).wait()
      pltpu.make_async_copy(ones, acc.at[dst_v.at[2 * t + 1]], s1).wait()
      return 0
    lax.fori_loop(0, cnt // 2, body, 0)

    plsc.subcore_barrier()
    pltpu.sync_copy(acc.at[pl.ds(sid * RPT, RPT)],
                    out.at[cid, pl.ds(sid * RPT, RPT)])

  _SC_CACHE["deg"] = degk
  return degk


def _sc_deg(dstg):
  return _make_sc_deg()(dstg)


def _l0_body(x_ref, p0_ref, p1_ref, d0_ref, d1_ref, ws_ref, wn_ref, b_ref,
             h_ref, inv_ref):
  s = p0_ref[...] + p1_ref[...]
  deg = d0_ref[:, :1] + d1_ref[:, :1]
  inv = 1.0 / jnp.maximum(deg, 1.0)
  inv_ref[...] = jnp.broadcast_to(inv, (BLK, 128))
  hn = s * inv
  h = jnp.dot(x_ref[...], ws_ref[...], preferred_element_type=jnp.float32)
  h = h + jnp.dot(hn, wn_ref[...], preferred_element_type=jnp.float32)
  h = h + b_ref[...]
  h_ref[...] = jnp.maximum(h, 0.0)


def _layer0(xp, p0, p1, d0, d1, Ws0, Wn0, b0):
  return pl.pallas_call(
      _l0_body,
      grid=(GRID,),
      in_specs=[
          pl.BlockSpec((BLK, 128), lambda i: (i, 0)),
          pl.BlockSpec((BLK, 128), lambda i: (i, 0)),
          pl.BlockSpec((BLK, 128), lambda i: (i, 0)),
          pl.BlockSpec((BLK, 16), lambda i: (i, 0)),
          pl.BlockSpec((BLK, 16), lambda i: (i, 0)),
          pl.BlockSpec((128, 256), lambda i: (0, 0)),
          pl.BlockSpec((128, 256), lambda i: (0, 0)),
          pl.BlockSpec((1, 256), lambda i: (0, 0)),
      ],
      out_specs=[
          pl.BlockSpec((BLK, 256), lambda i: (i, 0)),
          pl.BlockSpec((BLK, 128), lambda i: (i, 0)),
      ],
      out_shape=[
          jax.ShapeDtypeStruct((NP, 256), jnp.float32),
          jax.ShapeDtypeStruct((NP, 128), jnp.float32),
      ],
  )(xp, p0, p1, d0, d1, Ws0, Wn0, b0)


def _l1_body(h1_ref, qlo0_ref, qlo1_ref, qhi0_ref, qhi1_ref, inv_ref,
             ws1_ref, wn1a_ref, wn1b_ref, b1_ref, ws2_ref, wn2_ref, b2_ref,
             m2_ref, z_ref):
  inv = inv_ref[:, :1]
  slo = (qlo0_ref[...] + qlo1_ref[...]) * inv
  shi = (qhi0_ref[...] + qhi1_ref[...]) * inv
  h = jnp.dot(h1_ref[...], ws1_ref[...], preferred_element_type=jnp.float32)
  h = h + jnp.dot(slo, wn1a_ref[...], preferred_element_type=jnp.float32)
  h = h + jnp.dot(shi, wn1b_ref[...], preferred_element_type=jnp.float32)
  h = jnp.maximum(h + b1_ref[...], 0.0)
  m2_ref[...] = jnp.dot(h, wn2_ref[...], preferred_element_type=jnp.float32)
  z_ref[...] = jnp.dot(h, ws2_ref[...],
                       preferred_element_type=jnp.float32) + b2_ref[...]


def _layer1(h1, qlo0, qlo1, qhi0, qhi1, inv2d, Ws1, Wn1a, Wn1b, b1,
            Ws2p, Wn2p, b2p):
  return pl.pallas_call(
      _l1_body,
      grid=(GRID,),
      in_specs=[
          pl.BlockSpec((BLK, 256), lambda i: (i, 0)),
          pl.BlockSpec((BLK, 128), lambda i: (i, 0)),
          pl.BlockSpec((BLK, 128), lambda i: (i, 0)),
          pl.BlockSpec((BLK, 128), lambda i: (i, 0)),
          pl.BlockSpec((BLK, 128), lambda i: (i, 0)),
          pl.BlockSpec((BLK, 128), lambda i: (i, 0)),
          pl.BlockSpec((256, 256), lambda i: (0, 0)),
          pl.BlockSpec((128, 256), lambda i: (0, 0)),
          pl.BlockSpec((128, 256), lambda i: (0, 0)),
          pl.BlockSpec((1, 256), lambda i: (0, 0)),
          pl.BlockSpec((256, 48), lambda i: (0, 0)),
          pl.BlockSpec((256, 48), lambda i: (0, 0)),
          pl.BlockSpec((1, 48), lambda i: (0, 0)),
      ],
      out_specs=[
          pl.BlockSpec((BLK, 48), lambda i: (i, 0)),
          pl.BlockSpec((BLK, 48), lambda i: (i, 0)),
      ],
      out_shape=[
          jax.ShapeDtypeStruct((NP, 48), jnp.float32),
          jax.ShapeDtypeStruct((NP, 48), jnp.float32),
      ],
  )(h1, qlo0, qlo1, qhi0, qhi1, inv2d, Ws1, Wn1a, Wn1b, b1, Ws2p, Wn2p, b2p)


def _fin_body(z_ref, r0_ref, r1_ref, inv_ref, out_ref):
  s = (r0_ref[:N, :47] + r1_ref[:N, :47]) * inv_ref[:N, :1]
  out_ref[...] = z_ref[:N, :47] + s


def _final(z, r0, r1, inv2d):
  return pl.pallas_call(
      _fin_body,
      grid=(1,),
      in_specs=[
          pl.BlockSpec((NP, 48), lambda i: (0, 0)),
          pl.BlockSpec((NP, 48), lambda i: (0, 0)),
          pl.BlockSpec((NP, 48), lambda i: (0, 0)),
          pl.BlockSpec((NP, 128), lambda i: (0, 0)),
      ],
      out_specs=pl.BlockSpec((N, 47), lambda i: (0, 0)),
      out_shape=jax.ShapeDtypeStruct((N, 47), jnp.float32),
  )(z, r0, r1, inv2d)


def kernel(x, edge_index, W_self0, W_neigh0, b0, W_self1, W_neigh1, b1,
           W_self2, W_neigh2, b2):
  src = edge_index[0]
  dst = edge_index[1]
  srcg = jnp.concatenate(
      [src, jnp.zeros((NCH_PAD * CH - E,), jnp.int32)]).reshape(NCH_PAD, CH)
  dstg = jnp.concatenate(
      [dst, jnp.full((NCH_PAD * CH - E,), N, jnp.int32)]).reshape(
          NCH_PAD, CH)

  xp = jnp.pad(x, ((0, NP - N), (0, 0)))
  perm = jnp.asarray(PERM, jnp.int32)

  d = _sc_deg(dstg)
  p = _sc_segsum_bf(_pack_bf(xp), srcg, dstg)
  h1, inv2d = _layer0(xp, p[0], p[1], d[0], d[1], W_self0, W_neigh0[perm],
                      b0.reshape(1, 256))

  qlo = _sc_segsum_bf(_pack_bf(h1[:, :128]), srcg, dstg)
  qhi = _sc_segsum_bf(_pack_bf(h1[:, 128:]), srcg, dstg)

  Ws2p = jnp.pad(W_self2, ((0, 0), (0, 1)))
  Wn2p = jnp.pad(W_neigh2, ((0, 0), (0, 1)))
  b2p = jnp.pad(b2, ((0, 1),)).reshape(1, 48)
  m2, z = _layer1(h1, qlo[0], qlo[1], qhi[0], qhi[1], inv2d,
                  W_self1, W_neigh1[:128][perm], W_neigh1[128:][perm],
                  b1.reshape(1, 256), Ws2p, Wn2p, b2p)

  r = _sc_segsum(m2, srcg, dstg)
  return _final(z, r[0], r[1], inv2d)


# speedup vs baseline: 1.1008x; 1.1008x over previous
import functools

import jax
import jax.numpy as jnp
from jax import lax
from jax.experimental import pallas as pl
from jax.experimental.pallas import tpu as pltpu
from jax.experimental.pallas import tpu_sc as plsc

N = 10000
NP = 10240
E = 320000
CH = 128
NWORK = 32
CPW = 80
NCH = NWORK * CPW
EP = NCH * CH
CPW0 = 80
CPW1 = 2 * CPW - CPW0
CPWMAX = max(CPW0, CPW1)
BF0 = 88
BF1 = 2 * CPW - BF0
BFMAX = max(BF0, BF1)
NCH_PAD = 16 * CPW0 + 16 * CPW1 + BFMAX
SUP = 8
RPT = NP // 16
BLK = 1024
GRID = NP // BLK

_SC_PARAMS = pltpu.CompilerParams(use_tc_tiling_on_sc=False)


def _mesh():
  return plsc.VectorSubcoreMesh(core_axis_name="c", subcore_axis_name="s")


_SC_CACHE = {}


def _make_sc_segsum(D):
  if D in _SC_CACHE:
    return _SC_CACHE[D]

  @functools.partial(
      pl.kernel,
      mesh=_mesh(),
      compiler_params=_SC_PARAMS,
      out_type=jax.ShapeDtypeStruct((2, NP, D), jnp.float32),
      scratch_types=[
          pltpu.VMEM((CPWMAX, CH), jnp.int32),
          pltpu.VMEM((CPWMAX, CH), jnp.int32),
          pltpu.VMEM((CH, D), jnp.float32),
          pltpu.VMEM_SHARED((NP, D), jnp.float32),
          pltpu.SemaphoreType.DMA,
      ],
  )
  def seg(table, srcg, dstg, out, src_v, dst_v, rows_v, acc, sem):
    cid = lax.axis_index("c")
    sid = lax.axis_index("s")
    wbase = jnp.where(cid == 0, sid * CPW0, 16 * CPW0 + sid * CPW1)
    cnt = jnp.where(cid == 0, CPW0, CPW1)

    with jax.named_scope("zero"):
      def zrow(r, _):
        def zcol(c, _):
          rows_v[r, pl.ds(c * 16, 16)] = jnp.zeros((16,), jnp.float32)
          return 0
        return lax.fori_loop(0, D // 16, zcol, 0)
      lax.fori_loop(0, CH, zrow, 0)
      for k in range(RPT // CH):
        pltpu.sync_copy(rows_v, acc.at[pl.ds(sid * RPT + k * CH, CH)])
      plsc.subcore_barrier()

    with jax.named_scope("stage"):
      pltpu.sync_copy(srcg.at[pl.ds(wbase, CPWMAX)], src_v)
      pltpu.sync_copy(dstg.at[pl.ds(wbase, CPWMAX)], dst_v)

    with jax.named_scope("edges"):
      def body(j, _):
        pltpu.async_copy(table.at[src_v.at[j]], rows_v, sem).wait()
        pltpu.sync_copy(rows_v, acc.at[dst_v.at[j]], add=True)
        return 0
      lax.fori_loop(0, cnt, body, 0)
      plsc.subcore_barrier()

    with jax.named_scope("wb"):
      pltpu.sync_copy(acc.at[pl.ds(sid * RPT, RPT)],
                      out.at[cid, pl.ds(sid * RPT, RPT)])

  _SC_CACHE[D] = seg
  return seg


def _sc_segsum(table, srcg, dstg):
  return _make_sc_segsum(table.shape[1])(table, srcg, dstg)


PERM = sum([[32 * g + 2 * k for k in range(16)]
            + [32 * g + 2 * k + 1 for k in range(16)] for g in range(4)], [])


def _make_sc_segsum_bf():
  if "bf" in _SC_CACHE:
    return _SC_CACHE["bf"]

  @functools.partial(
      pl.kernel,
      mesh=_mesh(),
      compiler_params=_SC_PARAMS,
      out_type=jax.ShapeDtypeStruct((2, NP, 128), jnp.float32),
      scratch_types=[
          pltpu.VMEM((BFMAX + 1, CH), jnp.int32),
          pltpu.VMEM((SUP, CH), jnp.int32),
          pltpu.VMEM((CH, 64), jnp.int32),
          pltpu.VMEM((CH, 64), jnp.int32),
          pltpu.VMEM((CH, 128), jnp.float32),
          pltpu.VMEM_SHARED((NP, 128), jnp.float32),
          pltpu.SemaphoreType.DMA,
          pltpu.SemaphoreType.DMA,
      ],
  )
  def segbf(table, srcg, dstg, out, src_v, dst_v, bf0, bf1, rows_f, acc,
            g0, g1):
    cid = lax.axis_index("c")
    sid = lax.axis_index("s")
    wbase = jnp.where(cid == 0, sid * BF0, 16 * BF0 + sid * BF1)
    nsup = jnp.where(cid == 0, BF0 // SUP, BF1 // SUP)
    bf = [bf0, bf1]
    gsem = [g0, g1]

    def issue_g(j, b):
      pltpu.async_copy(table.at[src_v.at[j]], bf[b], gsem[b])

    def wait_g(j, b):
      pltpu.make_async_copy(table.at[src_v.at[j]], bf[b], gsem[b]).wait()

    with jax.named_scope("zero"):
      def zrow(r, _):
        def zcol(c, _):
          rows_f[r, pl.ds(c * 16, 16)] = jnp.zeros((16,), jnp.float32)
          return 0
        return lax.fori_loop(0, 8, zcol, 0)
      lax.fori_loop(0, CH, zrow, 0)
      for k in range(RPT // CH):
        pltpu.sync_copy(rows_f, acc.at[pl.ds(sid * RPT + k * CH, CH)])
      for c in range(CH // 16):
        src_v[BFMAX, pl.ds(c * 16, 16)] = jnp.zeros((16,), jnp.int32)
      plsc.subcore_barrier()

    with jax.named_scope("stage"):
      pltpu.sync_copy(srcg.at[pl.ds(wbase, BFMAX)], src_v.at[pl.ds(0, BFMAX)])

    with jax.named_scope("edges"):
      issue_g(0, 0)

      def super_body(t, _):
        pltpu.sync_copy(dstg.at[pl.ds(wbase + t * SUP, SUP)], dst_v)
        for k in range(SUP):
          j = t * SUP + k
          b = k % 2
          wait_g(j, b)
          issue_g(j + 1, 1 - b)

          def crow(r4, _):
            for rr in range(4):
              r = r4 * 4 + rr
              for g in range(4):
                v = bf[b][r, pl.ds(g * 16, 16)]
                lo = lax.bitcast_convert_type(v << 16, jnp.float32)
                hi = lax.bitcast_convert_type(v & jnp.int32(-65536),
                                              jnp.float32)
                rows_f[r, pl.ds(g * 32, 16)] = lo
                rows_f[r, pl.ds(g * 32 + 16, 16)] = hi
            return 0
          lax.fori_loop(0, CH // 4, crow, 0)

          pltpu.sync_copy(rows_f, acc.at[dst_v.at[k]], add=True)
        return 0
      lax.fori_loop(0, nsup, super_body, 0)

      wait_g(nsup * SUP, 0)
      plsc.subcore_barrier()

    with jax.named_scope("wb"):
      pltpu.sync_copy(acc.at[pl.ds(sid * RPT, RPT)],
                      out.at[cid, pl.ds(sid * RPT, RPT)])

  _SC_CACHE["bf"] = segbf
  return segbf


def _sc_segsum_bf(table_i32, srcg, dstg):
  return _make_sc_segsum_bf()(table_i32, srcg, dstg)


def _pack_bf(a):
  return lax.bitcast_convert_type(
      a.astype(jnp.bfloat16).reshape(NP, 64, 2), jnp.int32)


def _make_sc_deg():
  if "deg" in _SC_CACHE:
    return _SC_CACHE["deg"]

  @functools.partial(
      pl.kernel,
      mesh=_mesh(),
      compiler_params=_SC_PARAMS,
      out_type=jax.ShapeDtypeStruct((2, NP, 16), jnp.float32),
      scratch_types=[
          pltpu.VMEM((CPWMAX, CH), jnp.int32),
          pltpu.VMEM((CH, 16), jnp.float32),
          pltpu.VMEM_SHARED((NP, 16), jnp.float32),
          pltpu.SemaphoreType.DMA,
          pltpu.SemaphoreType.DMA,
      ],
  )
  def degk(dstg, out, dst_v, ones, acc, s0, s1):
    cid = lax.axis_index("c")
    sid = lax.axis_index("s")
    wbase = jnp.where(cid == 0, sid * CPW0, 16 * CPW0 + sid * CPW1)
    cnt = jnp.where(cid == 0, CPW0, CPW1)

    def fill(val):
      def frow(r, _):
        ones[r, pl.ds(0, 16)] = jnp.full((16,), val, jnp.float32)
        return 0
      lax.fori_loop(0, CH, frow, 0)

    fill(0.0)
    for k in range(RPT // CH):
      pltpu.sync_copy(ones, acc.at[pl.ds(sid * RPT + k * CH, CH)])
    plsc.subcore_barrier()
    fill(1.0)

    pltpu.sync_copy(dstg.at[pl.ds(wbase, CPWMAX)], dst_v)

    def body(t, _):
      pltpu.async_copy(ones, acc.at[dst_v.at[2 * t]], s0, add=True)
      pltpu.async_copy(ones, acc.at[dst_v.at[2 * t + 1]], s1, add=True)
      pltpu.make_async_copy(ones, acc.at[dst_v.at[2 * t]], s0).wait()
      pltpu.make_async_copy(ones, acc.at[dst_v.at[2 * t + 1]], s1).wait()
      return 0
    lax.fori_loop(0, cnt // 2, body, 0)

    plsc.subcore_barrier()
    pltpu.sync_copy(acc.at[pl.ds(sid * RPT, RPT)],
                    out.at[cid, pl.ds(sid * RPT, RPT)])

  _SC_CACHE["deg"] = degk
  return degk


def _sc_deg(dstg):
  return _make_sc_deg()(dstg)


def _l0_body(x_ref, p0_ref, p1_ref, d0_ref, d1_ref, ws_ref, wn_ref, b_ref,
             h_ref, inv_ref):
  s = p0_ref[...] + p1_ref[...]
  deg = d0_ref[:, :1] + d1_ref[:, :1]
  inv = 1.0 / jnp.maximum(deg, 1.0)
  inv_ref[...] = jnp.broadcast_to(inv, (BLK, 128))
  hn = s * inv
  h = jnp.dot(x_ref[...], ws_ref[...], preferred_element_type=jnp.float32)
  h = h + jnp.dot(hn, wn_ref[...], preferred_element_type=jnp.float32)
  h = h + b_ref[...]
  h_ref[...] = jnp.maximum(h, 0.0)


def _layer0(xp, p0, p1, d0, d1, Ws0, Wn0, b0):
  return pl.pallas_call(
      _l0_body,
      grid=(GRID,),
      in_specs=[
          pl.BlockSpec((BLK, 128), lambda i: (i, 0)),
          pl.BlockSpec((BLK, 128), lambda i: (i, 0)),
          pl.BlockSpec((BLK, 128), lambda i: (i, 0)),
          pl.BlockSpec((BLK, 16), lambda i: (i, 0)),
          pl.BlockSpec((BLK, 16), lambda i: (i, 0)),
          pl.BlockSpec((128, 256), lambda i: (0, 0)),
          pl.BlockSpec((128, 256), lambda i: (0, 0)),
          pl.BlockSpec((1, 256), lambda i: (0, 0)),
      ],
      out_specs=[
          pl.BlockSpec((BLK, 256), lambda i: (i, 0)),
          pl.BlockSpec((BLK, 128), lambda i: (i, 0)),
      ],
      out_shape=[
          jax.ShapeDtypeStruct((NP, 256), jnp.float32),
          jax.ShapeDtypeStruct((NP, 128), jnp.float32),
      ],
  )(xp, p0, p1, d0, d1, Ws0, Wn0, b0)


def _l1_body(h1_ref, qlo0_ref, qlo1_ref, qhi0_ref, qhi1_ref, inv_ref,
             ws1_ref, wn1a_ref, wn1b_ref, b1_ref, ws2_ref, wn2_ref, b2_ref,
             m2_ref, z_ref):
  inv = inv_ref[:, :1]
  slo = (qlo0_ref[...] + qlo1_ref[...]) * inv
  shi = (qhi0_ref[...] + qhi1_ref[...]) * inv
  h = jnp.dot(h1_ref[...], ws1_ref[...], preferred_element_type=jnp.float32)
  h = h + jnp.dot(slo, wn1a_ref[...], preferred_element_type=jnp.float32)
  h = h + jnp.dot(shi, wn1b_ref[...], preferred_element_type=jnp.float32)
  h = jnp.maximum(h + b1_ref[...], 0.0)
  m2_ref[...] = jnp.dot(h, wn2_ref[...], preferred_element_type=jnp.float32)
  z_ref[...] = jnp.dot(h, ws2_ref[...],
                       preferred_element_type=jnp.float32) + b2_ref[...]


def _layer1(h1, qlo0, qlo1, qhi0, qhi1, inv2d, Ws1, Wn1a, Wn1b, b1,
            Ws2p, Wn2p, b2p):
  return pl.pallas_call(
      _l1_body,
      grid=(GRID,),
      in_specs=[
          pl.BlockSpec((BLK, 256), lambda i: (i, 0)),
          pl.BlockSpec((BLK, 128), lambda i: (i, 0)),
          pl.BlockSpec((BLK, 128), lambda i: (i, 0)),
          pl.BlockSpec((BLK, 128), lambda i: (i, 0)),
          pl.BlockSpec((BLK, 128), lambda i: (i, 0)),
          pl.BlockSpec((BLK, 128), lambda i: (i, 0)),
          pl.BlockSpec((256, 256), lambda i: (0, 0)),
          pl.BlockSpec((128, 256), lambda i: (0, 0)),
          pl.BlockSpec((128, 256), lambda i: (0, 0)),
          pl.BlockSpec((1, 256), lambda i: (0, 0)),
          pl.BlockSpec((256, 48), lambda i: (0, 0)),
          pl.BlockSpec((256, 48), lambda i: (0, 0)),
          pl.BlockSpec((1, 48), lambda i: (0, 0)),
      ],
      out_specs=[
          pl.BlockSpec((BLK, 48), lambda i: (i, 0)),
          pl.BlockSpec((BLK, 48), lambda i: (i, 0)),
      ],
      out_shape=[
          jax.ShapeDtypeStruct((NP, 48), jnp.float32),
          jax.ShapeDtypeStruct((NP, 48), jnp.float32),
      ],
  )(h1, qlo0, qlo1, qhi0, qhi1, inv2d, Ws1, Wn1a, Wn1b, b1, Ws2p, Wn2p, b2p)


def _fin_body(z_ref, r0_ref, r1_ref, inv_ref, out_ref):
  s = (r0_ref[:N, :47] + r1_ref[:N, :47]) * inv_ref[:N, :1]
  out_ref[...] = z_ref[:N, :47] + s


def _final(z, r0, r1, inv2d):
  return pl.pallas_call(
      _fin_body,
      grid=(1,),
      in_specs=[
          pl.BlockSpec((NP, 48), lambda i: (0, 0)),
          pl.BlockSpec((NP, 48), lambda i: (0, 0)),
          pl.BlockSpec((NP, 48), lambda i: (0, 0)),
          pl.BlockSpec((NP, 128), lambda i: (0, 0)),
      ],
      out_specs=pl.BlockSpec((N, 47), lambda i: (0, 0)),
      out_shape=jax.ShapeDtypeStruct((N, 47), jnp.float32),
  )(z, r0, r1, inv2d)


def kernel(x, edge_index, W_self0, W_neigh0, b0, W_self1, W_neigh1, b1,
           W_self2, W_neigh2, b2):
  src = edge_index[0]
  dst = edge_index[1]
  srcg = jnp.concatenate(
      [src, jnp.zeros((NCH_PAD * CH - E,), jnp.int32)]).reshape(NCH_PAD, CH)
  dstg = jnp.concatenate(
      [dst, jnp.full((NCH_PAD * CH - E,), N, jnp.int32)]).reshape(
          NCH_PAD, CH)

  xp = jnp.pad(x, ((0, NP - N), (0, 0)))
  perm = jnp.asarray(PERM, jnp.int32)

  d = _sc_deg(dstg)
  p = _sc_segsum_bf(_pack_bf(xp), srcg, dstg)
  h1, inv2d = _layer0(xp, p[0], p[1], d[0], d[1], W_self0, W_neigh0[perm],
                      b0.reshape(1, 256))

  qlo = _sc_segsum_bf(_pack_bf(h1[:, :128]), srcg, dstg)
  qhi = _sc_segsum_bf(_pack_bf(h1[:, 128:]), srcg, dstg)

  Ws2p = jnp.pad(W_self2, ((0, 0), (0, 1)))
  Wn2p = jnp.pad(W_neigh2, ((0, 0), (0, 1)))
  b2p = jnp.pad(b2, ((0, 1),)).reshape(1, 48)
  m2, z = _layer1(h1, qlo[0], qlo[1], qhi[0], qhi[1], inv2d,
                  W_self1, W_neigh1[:128][perm], W_neigh1[128:][perm],
                  b1.reshape(1, 256), Ws2p, Wn2p, b2p)

  r = _sc_segsum(m2, srcg, dstg)
  return _final(z, r[0], r[1], inv2d)


# speedup vs baseline: 1.1252x; 1.0222x over previous
import functools

import jax
import jax.numpy as jnp
from jax import lax
from jax.experimental import pallas as pl
from jax.experimental.pallas import tpu as pltpu
from jax.experimental.pallas import tpu_sc as plsc

N = 10000
NP = 10240
E = 320000
CH = 128
NWORK = 32
CPW = 80
NCH = NWORK * CPW
EP = NCH * CH
CPW0 = 108
CPW1 = 2 * CPW - CPW0
CPWMAX = max(CPW0, CPW1)
BF0 = 88
BF1 = 2 * CPW - BF0
BFMAX = max(BF0, BF1)
NCH_PAD = 16 * CPW0 + 16 * CPW1 + max(CPWMAX, BFMAX)
SUP = 8
RPT = NP // 16
BLK = 1024
GRID = NP // BLK

_SC_PARAMS = pltpu.CompilerParams(use_tc_tiling_on_sc=False)


def _mesh():
  return plsc.VectorSubcoreMesh(core_axis_name="c", subcore_axis_name="s")


_SC_CACHE = {}


def _make_sc_segsum(D):
  if D in _SC_CACHE:
    return _SC_CACHE[D]

  @functools.partial(
      pl.kernel,
      mesh=_mesh(),
      compiler_params=_SC_PARAMS,
      out_type=jax.ShapeDtypeStruct((2, NP, D), jnp.float32),
      scratch_types=[
          pltpu.VMEM((CPWMAX, CH), jnp.int32),
          pltpu.VMEM((CPWMAX, CH), jnp.int32),
          pltpu.VMEM((CH, D), jnp.float32),
          pltpu.VMEM_SHARED((NP, D), jnp.float32),
          pltpu.SemaphoreType.DMA,
      ],
  )
  def seg(table, srcg, dstg, out, src_v, dst_v, rows_v, acc, sem):
    cid = lax.axis_index("c")
    sid = lax.axis_index("s")
    wbase = jnp.where(cid == 0, sid * CPW0, 16 * CPW0 + sid * CPW1)
    cnt = jnp.where(cid == 0, CPW0, CPW1)

    with jax.named_scope("zero"):
      def zrow(r, _):
        def zcol(c, _):
          rows_v[r, pl.ds(c * 16, 16)] = jnp.zeros((16,), jnp.float32)
          return 0
        return lax.fori_loop(0, D // 16, zcol, 0)
      lax.fori_loop(0, CH, zrow, 0)
      for k in range(RPT // CH):
        pltpu.sync_copy(rows_v, acc.at[pl.ds(sid * RPT + k * CH, CH)])
      plsc.subcore_barrier()

    with jax.named_scope("stage"):
      pltpu.sync_copy(srcg.at[pl.ds(wbase, CPWMAX)], src_v)
      pltpu.sync_copy(dstg.at[pl.ds(wbase, CPWMAX)], dst_v)

    with jax.named_scope("edges"):
      def body(j, _):
        pltpu.async_copy(table.at[src_v.at[j]], rows_v, sem).wait()
        pltpu.sync_copy(rows_v, acc.at[dst_v.at[j]], add=True)
        return 0
      lax.fori_loop(0, cnt, body, 0)
      plsc.subcore_barrier()

    with jax.named_scope("wb"):
      pltpu.sync_copy(acc.at[pl.ds(sid * RPT, RPT)],
                      out.at[cid, pl.ds(sid * RPT, RPT)])

  _SC_CACHE[D] = seg
  return seg


def _sc_segsum(table, srcg, dstg):
  return _make_sc_segsum(table.shape[1])(table, srcg, dstg)


PERM = sum([[32 * g + 2 * k for k in range(16)]
            + [32 * g + 2 * k + 1 for k in range(16)] for g in range(4)], [])


def _make_sc_segsum_bf():
  if "bf" in _SC_CACHE:
    return _SC_CACHE["bf"]

  @functools.partial(
      pl.kernel,
      mesh=_mesh(),
      compiler_params=_SC_PARAMS,
      out_type=jax.ShapeDtypeStruct((2, NP, 128), jnp.float32),
      scratch_types=[
          pltpu.VMEM((BFMAX + 1, CH), jnp.int32),
          pltpu.VMEM((SUP, CH), jnp.int32),
          pltpu.VMEM((CH, 64), jnp.int32),
          pltpu.VMEM((CH, 64), jnp.int32),
          pltpu.VMEM((CH, 128), jnp.float32),
          pltpu.VMEM_SHARED((NP, 128), jnp.float32),
          pltpu.SemaphoreType.DMA,
          pltpu.SemaphoreType.DMA,
      ],
  )
  def segbf(table, srcg, dstg, out, src_v, dst_v, bf0, bf1, rows_f, acc,
            g0, g1):
    cid = lax.axis_index("c")
    sid = lax.axis_index("s")
    wbase = jnp.where(cid == 0, sid * BF0, 16 * BF0 + sid * BF1)
    nsup = jnp.where(cid == 0, BF0 // SUP, BF1 // SUP)
    bf = [bf0, bf1]
    gsem = [g0, g1]

    def issue_g(j, b):
      pltpu.async_copy(table.at[src_v.at[j]], bf[b], gsem[b])

    def wait_g(j, b):
      pltpu.make_async_copy(table.at[src_v.at[j]], bf[b], gsem[b]).wait()

    with jax.named_scope("zero"):
      def zrow(r, _):
        def zcol(c, _):
          rows_f[r, pl.ds(c * 16, 16)] = jnp.zeros((16,), jnp.float32)
          return 0
        return lax.fori_loop(0, 8, zcol, 0)
      lax.fori_loop(0, CH, zrow, 0)
      for k in range(RPT // CH):
        pltpu.sync_copy(rows_f, acc.at[pl.ds(sid * RPT + k * CH, CH)])
      for c in range(CH // 16):
        src_v[BFMAX, pl.ds(c * 16, 16)] = jnp.zeros((16,), jnp.int32)
      plsc.subcore_barrier()

    with jax.named_scope("stage"):
      pltpu.sync_copy(srcg.at[pl.ds(wbase, BFMAX)], src_v.at[pl.ds(0, BFMAX)])

    with jax.named_scope("edges"):
      issue_g(0, 0)

      def super_body(t, _):
        pltpu.sync_copy(dstg.at[pl.ds(wbase + t * SUP, SUP)], dst_v)
        for k in range(SUP):
          j = t * SUP + k
          b = k % 2
          wait_g(j, b)
          issue_g(j + 1, 1 - b)

          def crow(r4, _):
            for rr in range(4):
              r = r4 * 4 + rr
              for g in range(4):
                v = bf[b][r, pl.ds(g * 16, 16)]
                lo = lax.bitcast_convert_type(v << 16, jnp.float32)
                hi = lax.bitcast_convert_type(v & jnp.int32(-65536),
                                              jnp.float32)
                rows_f[r, pl.ds(g * 32, 16)] = lo
                rows_f[r, pl.ds(g * 32 + 16, 16)] = hi
            return 0
          lax.fori_loop(0, CH // 4, crow, 0)

          pltpu.sync_copy(rows_f, acc.at[dst_v.at[k]], add=True)
        return 0
      lax.fori_loop(0, nsup, super_body, 0)

      wait_g(nsup * SUP, 0)
      plsc.subcore_barrier()

    with jax.named_scope("wb"):
      pltpu.sync_copy(acc.at[pl.ds(sid * RPT, RPT)],
                      out.at[cid, pl.ds(sid * RPT, RPT)])

  _SC_CACHE["bf"] = segbf
  return segbf


def _sc_segsum_bf(table_i32, srcg, dstg):
  return _make_sc_segsum_bf()(table_i32, srcg, dstg)


def _pack_bf(a):
  return lax.bitcast_convert_type(
      a.astype(jnp.bfloat16).reshape(NP, 64, 2), jnp.int32)


def _make_sc_deg():
  if "deg" in _SC_CACHE:
    return _SC_CACHE["deg"]

  @functools.partial(
      pl.kernel,
      mesh=_mesh(),
      compiler_params=_SC_PARAMS,
      out_type=jax.ShapeDtypeStruct((2, NP, 16), jnp.float32),
      scratch_types=[
          pltpu.VMEM((CPWMAX, CH), jnp.int32),
          pltpu.VMEM((CH, 16), jnp.float32),
          pltpu.VMEM_SHARED((NP, 16), jnp.float32),
          pltpu.SemaphoreType.DMA,
          pltpu.SemaphoreType.DMA,
      ],
  )
  def degk(dstg, out, dst_v, ones, acc, s0, s1):
    cid = lax.axis_index("c")
    sid = lax.axis_index("s")
    wbase = jnp.where(cid == 0, sid * CPW0, 16 * CPW0 + sid * CPW1)
    cnt = jnp.where(cid == 0, CPW0, CPW1)

    def fill(val):
      def frow(r, _):
        ones[r, pl.ds(0, 16)] = jnp.full((16,), val, jnp.float32)
        return 0
      lax.fori_loop(0, CH, frow, 0)

    fill(0.0)
    for k in range(RPT // CH):
      pltpu.sync_copy(ones, acc.at[pl.ds(sid * RPT + k * CH, CH)])
    plsc.subcore_barrier()
    fill(1.0)

    pltpu.sync_copy(dstg.at[pl.ds(wbase, CPWMAX)], dst_v)

    def body(t, _):
      pltpu.async_copy(ones, acc.at[dst_v.at[2 * t]], s0, add=True)
      pltpu.async_copy(ones, acc.at[dst_v.at[2 * t + 1]], s1, add=True)
      pltpu.make_async_copy(ones, acc.at[dst_v.at[2 * t]], s0).wait()
      pltpu.make_async_copy(ones, acc.at[dst_v.at[2 * t + 1]], s1).wait()
      return 0
    lax.fori_loop(0, cnt // 2, body, 0)

    plsc.subcore_barrier()
    pltpu.sync_copy(acc.at[pl.ds(sid * RPT, RPT)],
                    out.at[cid, pl.ds(sid * RPT, RPT)])

  _SC_CACHE["deg"] = degk
  return degk


def _sc_deg(dstg):
  return _make_sc_deg()(dstg)


def _l0_body(x_ref, p0_ref, p1_ref, d0_ref, d1_ref, ws_ref, wn_ref, b_ref,
             h_ref, inv_ref):
  s = p0_ref[...] + p1_ref[...]
  deg = d0_ref[:, :1] + d1_ref[:, :1]
  inv = 1.0 / jnp.maximum(deg, 1.0)
  inv_ref[...] = jnp.broadcast_to(inv, (BLK, 128))
  hn = s * inv
  h = jnp.dot(x_ref[...], ws_ref[...], preferred_element_type=jnp.float32)
  h = h + jnp.dot(hn, wn_ref[...], preferred_element_type=jnp.float32)
  h = h + b_ref[...]
  h_ref[...] = jnp.maximum(h, 0.0)


def _layer0(xp, p0, p1, d0, d1, Ws0, Wn0, b0):
  return pl.pallas_call(
      _l0_body,
      grid=(GRID,),
      in_specs=[
          pl.BlockSpec((BLK, 128), lambda i: (i, 0)),
          pl.BlockSpec((BLK, 128), lambda i: (i, 0)),
          pl.BlockSpec((BLK, 128), lambda i: (i, 0)),
          pl.BlockSpec((BLK, 16), lambda i: (i, 0)),
          pl.BlockSpec((BLK, 16), lambda i: (i, 0)),
          pl.BlockSpec((128, 256), lambda i: (0, 0)),
          pl.BlockSpec((128, 256), lambda i: (0, 0)),
          pl.BlockSpec((1, 256), lambda i: (0, 0)),
      ],
      out_specs=[
          pl.BlockSpec((BLK, 256), lambda i: (i, 0)),
          pl.BlockSpec((BLK, 128), lambda i: (i, 0)),
      ],
      out_shape=[
          jax.ShapeDtypeStruct((NP, 256), jnp.float32),
          jax.ShapeDtypeStruct((NP, 128), jnp.float32),
      ],
  )(xp, p0, p1, d0, d1, Ws0, Wn0, b0)


def _l1_body(h1_ref, qlo0_ref, qlo1_ref, qhi0_ref, qhi1_ref, inv_ref,
             ws1_ref, wn1a_ref, wn1b_ref, b1_ref, ws2_ref, wn2_ref, b2_ref,
             m2_ref, z_ref):
  inv = inv_ref[:, :1]
  slo = (qlo0_ref[...] + qlo1_ref[...]) * inv
  shi = (qhi0_ref[...] + qhi1_ref[...]) * inv
  h = jnp.dot(h1_ref[...], ws1_ref[...], preferred_element_type=jnp.float32)
  h = h + jnp.dot(slo, wn1a_ref[...], preferred_element_type=jnp.float32)
  h = h + jnp.dot(shi, wn1b_ref[...], preferred_element_type=jnp.float32)
  h = jnp.maximum(h + b1_ref[...], 0.0)
  m2_ref[...] = jnp.dot(h, wn2_ref[...], preferred_element_type=jnp.float32)
  z_ref[...] = jnp.dot(h, ws2_ref[...],
                       preferred_element_type=jnp.float32) + b2_ref[...]


def _layer1(h1, qlo0, qlo1, qhi0, qhi1, inv2d, Ws1, Wn1a, Wn1b, b1,
            Ws2p, Wn2p, b2p):
  return pl.pallas_call(
      _l1_body,
      grid=(GRID,),
      in_specs=[
          pl.BlockSpec((BLK, 256), lambda i: (i, 0)),
          pl.BlockSpec((BLK, 128), lambda i: (i, 0)),
          pl.BlockSpec((BLK, 128), lambda i: (i, 0)),
          pl.BlockSpec((BLK, 128), lambda i: (i, 0)),
          pl.BlockSpec((BLK, 128), lambda i: (i, 0)),
          pl.BlockSpec((BLK, 128), lambda i: (i, 0)),
          pl.BlockSpec((256, 256), lambda i: (0, 0)),
          pl.BlockSpec((128, 256), lambda i: (0, 0)),
          pl.BlockSpec((128, 256), lambda i: (0, 0)),
          pl.BlockSpec((1, 256), lambda i: (0, 0)),
          pl.BlockSpec((256, 48), lambda i: (0, 0)),
          pl.BlockSpec((256, 48), lambda i: (0, 0)),
          pl.BlockSpec((1, 48), lambda i: (0, 0)),
      ],
      out_specs=[
          pl.BlockSpec((BLK, 48), lambda i: (i, 0)),
          pl.BlockSpec((BLK, 48), lambda i: (i, 0)),
      ],
      out_shape=[
          jax.ShapeDtypeStruct((NP, 48), jnp.float32),
          jax.ShapeDtypeStruct((NP, 48), jnp.float32),
      ],
  )(h1, qlo0, qlo1, qhi0, qhi1, inv2d, Ws1, Wn1a, Wn1b, b1, Ws2p, Wn2p, b2p)


def _fin_body(z_ref, r0_ref, r1_ref, inv_ref, out_ref):
  s = (r0_ref[:N, :47] + r1_ref[:N, :47]) * inv_ref[:N, :1]
  out_ref[...] = z_ref[:N, :47] + s


def _final(z, r0, r1, inv2d):
  return pl.pallas_call(
      _fin_body,
      grid=(1,),
      in_specs=[
          pl.BlockSpec((NP, 48), lambda i: (0, 0)),
          pl.BlockSpec((NP, 48), lambda i: (0, 0)),
          pl.BlockSpec((NP, 48), lambda i: (0, 0)),
          pl.BlockSpec((NP, 128), lambda i: (0, 0)),
      ],
      out_specs=pl.BlockSpec((N, 47), lambda i: (0, 0)),
      out_shape=jax.ShapeDtypeStruct((N, 47), jnp.float32),
  )(z, r0, r1, inv2d)


def kernel(x, edge_index, W_self0, W_neigh0, b0, W_self1, W_neigh1, b1,
           W_self2, W_neigh2, b2):
  src = edge_index[0]
  dst = edge_index[1]
  srcg = jnp.concatenate(
      [src, jnp.zeros((NCH_PAD * CH - E,), jnp.int32)]).reshape(NCH_PAD, CH)
  dstg = jnp.concatenate(
      [dst, jnp.full((NCH_PAD * CH - E,), N, jnp.int32)]).reshape(
          NCH_PAD, CH)

  xp = jnp.pad(x, ((0, NP - N), (0, 0)))
  perm = jnp.asarray(PERM, jnp.int32)

  d = _sc_deg(dstg)
  p = _sc_segsum_bf(_pack_bf(xp), srcg, dstg)
  h1, inv2d = _layer0(xp, p[0], p[1], d[0], d[1], W_self0, W_neigh0[perm],
                      b0.reshape(1, 256))

  qlo = _sc_segsum_bf(_pack_bf(h1[:, :128]), srcg, dstg)
  qhi = _sc_segsum_bf(_pack_bf(h1[:, 128:]), srcg, dstg)

  Ws2p = jnp.pad(W_self2, ((0, 0), (0, 1)))
  Wn2p = jnp.pad(W_neigh2, ((0, 0), (0, 1)))
  b2p = jnp.pad(b2, ((0, 1),)).reshape(1, 48)
  m2, z = _layer1(h1, qlo[0], qlo[1], qhi[0], qhi[1], inv2d,
                  W_self1, W_neigh1[:128][perm], W_neigh1[128:][perm],
                  b1.reshape(1, 256), Ws2p, Wn2p, b2p)

  r = _sc_segsum(m2, srcg, dstg)
  return _final(z, r[0], r[1], inv2d)


# speedup vs baseline: 1.1560x; 1.0273x over previous
import functools

import jax
import jax.numpy as jnp
from jax import lax
from jax.experimental import pallas as pl
from jax.experimental.pallas import tpu as pltpu
from jax.experimental.pallas import tpu_sc as plsc

N = 10000
NP = 10240
E = 320000
CH = 128
NWORK = 32
CPW = 80
NCH = NWORK * CPW
EP = NCH * CH
CPW0 = 108
CPW1 = 2 * CPW - CPW0
CPWMAX = max(CPW0, CPW1)
BF0 = 88
BF1 = 2 * CPW - BF0
BFMAX = max(BF0, BF1)
NCH_PAD = 16 * CPW0 + 16 * CPW1 + max(CPWMAX, BFMAX)
SUP = 8
RPT = NP // 16
BLK = 1024
GRID = NP // BLK

_SC_PARAMS = pltpu.CompilerParams(use_tc_tiling_on_sc=False)


def _mesh():
  return plsc.VectorSubcoreMesh(core_axis_name="c", subcore_axis_name="s")


_SC_CACHE = {}


def _make_sc_segsum(D):
  if D in _SC_CACHE:
    return _SC_CACHE[D]

  @functools.partial(
      pl.kernel,
      mesh=_mesh(),
      compiler_params=_SC_PARAMS,
      out_type=jax.ShapeDtypeStruct((2, NP, D), jnp.float32),
      scratch_types=[
          pltpu.VMEM((CPWMAX, CH), jnp.int32),
          pltpu.VMEM((CPWMAX, CH), jnp.int32),
          pltpu.VMEM((CH, D), jnp.float32),
          pltpu.VMEM_SHARED((NP, D), jnp.float32),
          pltpu.SemaphoreType.DMA,
      ],
  )
  def seg(table, srcg, dstg, out, src_v, dst_v, rows_v, acc, sem):
    cid = lax.axis_index("c")
    sid = lax.axis_index("s")
    wbase = jnp.where(cid == 0, sid * CPW0, 16 * CPW0 + sid * CPW1)
    cnt = jnp.where(cid == 0, CPW0, CPW1)

    with jax.named_scope("zero"):
      def zrow(r, _):
        def zcol(c, _):
          rows_v[r, pl.ds(c * 16, 16)] = jnp.zeros((16,), jnp.float32)
          return 0
        return lax.fori_loop(0, D // 16, zcol, 0)
      lax.fori_loop(0, CH, zrow, 0)
      for k in range(RPT // CH):
        pltpu.sync_copy(rows_v, acc.at[pl.ds(sid * RPT + k * CH, CH)])
      plsc.subcore_barrier()

    with jax.named_scope("stage"):
      pltpu.sync_copy(srcg.at[pl.ds(wbase, CPWMAX)], src_v)
      pltpu.sync_copy(dstg.at[pl.ds(wbase, CPWMAX)], dst_v)

    with jax.named_scope("edges"):
      def body(j, _):
        pltpu.async_copy(table.at[src_v.at[j]], rows_v, sem).wait()
        pltpu.sync_copy(rows_v, acc.at[dst_v.at[j]], add=True)
        return 0
      lax.fori_loop(0, cnt, body, 0)
      plsc.subcore_barrier()

    with jax.named_scope("wb"):
      pltpu.sync_copy(acc.at[pl.ds(sid * RPT, RPT)],
                      out.at[cid, pl.ds(sid * RPT, RPT)])

  _SC_CACHE[D] = seg
  return seg


def _sc_segsum(table, srcg, dstg):
  return _make_sc_segsum(table.shape[1])(table, srcg, dstg)


def _perm(D):
  return sum([[32 * g + 2 * k for k in range(16)]
              + [32 * g + 2 * k + 1 for k in range(16)]
              for g in range(D // 32)], [])


PERM = _perm(128)
INVPERM64 = [0] * 64
for _j, _f in enumerate(_perm(64)):
  INVPERM64[_f] = _j


def _make_sc_segsum_bf(D):
  if ("bf", D) in _SC_CACHE:
    return _SC_CACHE[("bf", D)]

  @functools.partial(
      pl.kernel,
      mesh=_mesh(),
      compiler_params=_SC_PARAMS,
      out_type=jax.ShapeDtypeStruct((2, NP, D), jnp.float32),
      scratch_types=[
          pltpu.VMEM((BFMAX + 1, CH), jnp.int32),
          pltpu.VMEM((SUP, CH), jnp.int32),
          pltpu.VMEM((CH, D // 2), jnp.int32),
          pltpu.VMEM((CH, D // 2), jnp.int32),
          pltpu.VMEM((CH, D), jnp.float32),
          pltpu.VMEM_SHARED((NP, D), jnp.float32),
          pltpu.SemaphoreType.DMA,
          pltpu.SemaphoreType.DMA,
      ],
  )
  def segbf(table, srcg, dstg, out, src_v, dst_v, bf0, bf1, rows_f, acc,
            g0, g1):
    cid = lax.axis_index("c")
    sid = lax.axis_index("s")
    wbase = jnp.where(cid == 0, sid * BF0, 16 * BF0 + sid * BF1)
    nsup = jnp.where(cid == 0, BF0 // SUP, BF1 // SUP)
    bf = [bf0, bf1]
    gsem = [g0, g1]

    def issue_g(j, b):
      pltpu.async_copy(table.at[src_v.at[j]], bf[b], gsem[b])

    def wait_g(j, b):
      pltpu.make_async_copy(table.at[src_v.at[j]], bf[b], gsem[b]).wait()

    with jax.named_scope("zero"):
      def zrow(r, _):
        def zcol(c, _):
          rows_f[r, pl.ds(c * 16, 16)] = jnp.zeros((16,), jnp.float32)
          return 0
        return lax.fori_loop(0, D // 16, zcol, 0)
      lax.fori_loop(0, CH, zrow, 0)
      for k in range(RPT // CH):
        pltpu.sync_copy(rows_f, acc.at[pl.ds(sid * RPT + k * CH, CH)])
      for c in range(CH // 16):
        src_v[BFMAX, pl.ds(c * 16, 16)] = jnp.zeros((16,), jnp.int32)
      plsc.subcore_barrier()

    with jax.named_scope("stage"):
      pltpu.sync_copy(srcg.at[pl.ds(wbase, BFMAX)], src_v.at[pl.ds(0, BFMAX)])

    with jax.named_scope("edges"):
      issue_g(0, 0)

      def super_body(t, _):
        pltpu.sync_copy(dstg.at[pl.ds(wbase + t * SUP, SUP)], dst_v)
        for k in range(SUP):
          j = t * SUP + k
          b = k % 2
          wait_g(j, b)
          issue_g(j + 1, 1 - b)

          def crow(r4, _):
            for rr in range(4):
              r = r4 * 4 + rr
              for g in range(D // 32):
                v = bf[b][r, pl.ds(g * 16, 16)]
                lo = lax.bitcast_convert_type(v << 16, jnp.float32)
                hi = lax.bitcast_convert_type(v & jnp.int32(-65536),
                                              jnp.float32)
                rows_f[r, pl.ds(g * 32, 16)] = lo
                rows_f[r, pl.ds(g * 32 + 16, 16)] = hi
            return 0
          lax.fori_loop(0, CH // 4, crow, 0)

          pltpu.sync_copy(rows_f, acc.at[dst_v.at[k]], add=True)
        return 0
      lax.fori_loop(0, nsup, super_body, 0)

      wait_g(nsup * SUP, 0)
      plsc.subcore_barrier()

    with jax.named_scope("wb"):
      pltpu.sync_copy(acc.at[pl.ds(sid * RPT, RPT)],
                      out.at[cid, pl.ds(sid * RPT, RPT)])

  _SC_CACHE[("bf", D)] = segbf
  return segbf


def _sc_segsum_bf(table_i32, srcg, dstg):
  return _make_sc_segsum_bf(2 * table_i32.shape[1])(table_i32, srcg, dstg)


def _pack_bf(a):
  d = a.shape[1]
  return lax.bitcast_convert_type(
      a.astype(jnp.bfloat16).reshape(NP, d // 2, 2), jnp.int32)


def _make_sc_deg():
  if "deg" in _SC_CACHE:
    return _SC_CACHE["deg"]

  @functools.partial(
      pl.kernel,
      mesh=_mesh(),
      compiler_params=_SC_PARAMS,
      out_type=jax.ShapeDtypeStruct((2, NP, 16), jnp.float32),
      scratch_types=[
          pltpu.VMEM((CPWMAX, CH), jnp.int32),
          pltpu.VMEM((CH, 16), jnp.float32),
          pltpu.VMEM_SHARED((NP, 16), jnp.float32),
          pltpu.SemaphoreType.DMA,
          pltpu.SemaphoreType.DMA,
      ],
  )
  def degk(dstg, out, dst_v, ones, acc, s0, s1):
    cid = lax.axis_index("c")
    sid = lax.axis_index("s")
    wbase = jnp.where(cid == 0, sid * CPW0, 16 * CPW0 + sid * CPW1)
    cnt = jnp.where(cid == 0, CPW0, CPW1)

    def fill(val):
      def frow(r, _):
        ones[r, pl.ds(0, 16)] = jnp.full((16,), val, jnp.float32)
        return 0
      lax.fori_loop(0, CH, frow, 0)

    fill(0.0)
    for k in range(RPT // CH):
      pltpu.sync_copy(ones, acc.at[pl.ds(sid * RPT + k * CH, CH)])
    plsc.subcore_barrier()
    fill(1.0)

    pltpu.sync_copy(dstg.at[pl.ds(wbase, CPWMAX)], dst_v)

    def body(t, _):
      pltpu.async_copy(ones, acc.at[dst_v.at[2 * t]], s0, add=True)
      pltpu.async_copy(ones, acc.at[dst_v.at[2 * t + 1]], s1, add=True)
      pltpu.make_async_copy(ones, acc.at[dst_v.at[2 * t]], s0).wait()
      pltpu.make_async_copy(ones, acc.at[dst_v.at[2 * t + 1]], s1).wait()
      return 0
    lax.fori_loop(0, cnt // 2, body, 0)

    plsc.subcore_barrier()
    pltpu.sync_copy(acc.at[pl.ds(sid * RPT, RPT)],
                    out.at[cid, pl.ds(sid * RPT, RPT)])

  _SC_CACHE["deg"] = degk
  return degk


def _sc_deg(dstg):
  return _make_sc_deg()(dstg)


def _l0_body(x_ref, p0_ref, p1_ref, d0_ref, d1_ref, ws_ref, wn_ref, b_ref,
             h_ref, inv_ref):
  s = p0_ref[...] + p1_ref[...]
  deg = d0_ref[:, :1] + d1_ref[:, :1]
  inv = 1.0 / jnp.maximum(deg, 1.0)
  inv_ref[...] = jnp.broadcast_to(inv, (BLK, 128))
  hn = s * inv
  h = jnp.dot(x_ref[...], ws_ref[...], preferred_element_type=jnp.float32)
  h = h + jnp.dot(hn, wn_ref[...], preferred_element_type=jnp.float32)
  h = h + b_ref[...]
  h_ref[...] = jnp.maximum(h, 0.0)


def _layer0(xp, p0, p1, d0, d1, Ws0, Wn0, b0):
  return pl.pallas_call(
      _l0_body,
      grid=(GRID,),
      in_specs=[
          pl.BlockSpec((BLK, 128), lambda i: (i, 0)),
          pl.BlockSpec((BLK, 128), lambda i: (i, 0)),
          pl.BlockSpec((BLK, 128), lambda i: (i, 0)),
          pl.BlockSpec((BLK, 16), lambda i: (i, 0)),
          pl.BlockSpec((BLK, 16), lambda i: (i, 0)),
          pl.BlockSpec((128, 256), lambda i: (0, 0)),
          pl.BlockSpec((128, 256), lambda i: (0, 0)),
          pl.BlockSpec((1, 256), lambda i: (0, 0)),
      ],
      out_specs=[
          pl.BlockSpec((BLK, 256), lambda i: (i, 0)),
          pl.BlockSpec((BLK, 128), lambda i: (i, 0)),
      ],
      out_shape=[
          jax.ShapeDtypeStruct((NP, 256), jnp.float32),
          jax.ShapeDtypeStruct((NP, 128), jnp.float32),
      ],
  )(xp, p0, p1, d0, d1, Ws0, Wn0, b0)


def _l1_body(h1_ref, qlo0_ref, qlo1_ref, qhi0_ref, qhi1_ref, inv_ref,
             ws1_ref, wn1a_ref, wn1b_ref, b1_ref, ws2_ref, wn2_ref, b2_ref,
             m2_ref, z_ref):
  inv = inv_ref[:, :1]
  slo = (qlo0_ref[...] + qlo1_ref[...]) * inv
  shi = (qhi0_ref[...] + qhi1_ref[...]) * inv
  h = jnp.dot(h1_ref[...], ws1_ref[...], preferred_element_type=jnp.float32)
  h = h + jnp.dot(slo, wn1a_ref[...], preferred_element_type=jnp.float32)
  h = h + jnp.dot(shi, wn1b_ref[...], preferred_element_type=jnp.float32)
  h = jnp.maximum(h + b1_ref[...], 0.0)
  m2_ref[...] = jnp.dot(h, wn2_ref[...], preferred_element_type=jnp.float32)
  z_ref[...] = jnp.dot(h, ws2_ref[...],
                       preferred_element_type=jnp.float32) + b2_ref[...]


def _layer1(h1, qlo0, qlo1, qhi0, qhi1, inv2d, Ws1, Wn1a, Wn1b, b1,
            Ws2p, Wn2p, b2p):
  return pl.pallas_call(
      _l1_body,
      grid=(GRID,),
      in_specs=[
          pl.BlockSpec((BLK, 256), lambda i: (i, 0)),
          pl.BlockSpec((BLK, 128), lambda i: (i, 0)),
          pl.BlockSpec((BLK, 128), lambda i: (i, 0)),
          pl.BlockSpec((BLK, 128), lambda i: (i, 0)),
          pl.BlockSpec((BLK, 128), lambda i: (i, 0)),
          pl.BlockSpec((BLK, 128), lambda i: (i, 0)),
          pl.BlockSpec((256, 256), lambda i: (0, 0)),
          pl.BlockSpec((128, 256), lambda i: (0, 0)),
          pl.BlockSpec((128, 256), lambda i: (0, 0)),
          pl.BlockSpec((1, 256), lambda i: (0, 0)),
          pl.BlockSpec((256, 48), lambda i: (0, 0)),
          pl.BlockSpec((256, 64), lambda i: (0, 0)),
          pl.BlockSpec((1, 48), lambda i: (0, 0)),
      ],
      out_specs=[
          pl.BlockSpec((BLK, 64), lambda i: (i, 0)),
          pl.BlockSpec((BLK, 48), lambda i: (i, 0)),
      ],
      out_shape=[
          jax.ShapeDtypeStruct((NP, 64), jnp.float32),
          jax.ShapeDtypeStruct((NP, 48), jnp.float32),
      ],
  )(h1, qlo0, qlo1, qhi0, qhi1, inv2d, Ws1, Wn1a, Wn1b, b1, Ws2p, Wn2p, b2p)


def _fin_body(z_ref, r0_ref, r1_ref, inv_ref, out_ref):
  s = (r0_ref[:N, :47] + r1_ref[:N, :47]) * inv_ref[:N, :1]
  out_ref[...] = z_ref[:N, :47] + s


def _final(z, r0, r1, inv2d):
  return pl.pallas_call(
      _fin_body,
      grid=(1,),
      in_specs=[
          pl.BlockSpec((NP, 48), lambda i: (0, 0)),
          pl.BlockSpec((NP, 64), lambda i: (0, 0)),
          pl.BlockSpec((NP, 64), lambda i: (0, 0)),
          pl.BlockSpec((NP, 128), lambda i: (0, 0)),
      ],
      out_specs=pl.BlockSpec((N, 47), lambda i: (0, 0)),
      out_shape=jax.ShapeDtypeStruct((N, 47), jnp.float32),
  )(z, r0, r1, inv2d)


def kernel(x, edge_index, W_self0, W_neigh0, b0, W_self1, W_neigh1, b1,
           W_self2, W_neigh2, b2):
  src = edge_index[0]
  dst = edge_index[1]
  srcg = jnp.concatenate(
      [src, jnp.zeros((NCH_PAD * CH - E,), jnp.int32)]).reshape(NCH_PAD, CH)
  dstg = jnp.concatenate(
      [dst, jnp.full((NCH_PAD * CH - E,), N, jnp.int32)]).reshape(
          NCH_PAD, CH)

  xp = jnp.pad(x, ((0, NP - N), (0, 0)))
  perm = jnp.asarray(PERM, jnp.int32)

  d = _sc_deg(dstg)
  p = _sc_segsum_bf(_pack_bf(xp), srcg, dstg)
  h1, inv2d = _layer0(xp, p[0], p[1], d[0], d[1], W_self0, W_neigh0[perm],
                      b0.reshape(1, 256))

  qlo = _sc_segsum_bf(_pack_bf(h1[:, :128]), srcg, dstg)
  qhi = _sc_segsum_bf(_pack_bf(h1[:, 128:]), srcg, dstg)

  Ws2p = jnp.pad(W_self2, ((0, 0), (0, 1)))
  Wn2p = jnp.pad(W_neigh2, ((0, 0), (0, 17)))[
      :, jnp.asarray(INVPERM64, jnp.int32)]
  b2p = jnp.pad(b2, ((0, 1),)).reshape(1, 48)
  m2, z = _layer1(h1, qlo[0], qlo[1], qhi[0], qhi[1], inv2d,
                  W_self1, W_neigh1[:128][perm], W_neigh1[128:][perm],
                  b1.reshape(1, 256), Ws2p, Wn2p, b2p)

  r = _sc_segsum_bf(_pack_bf(m2), srcg, dstg)
  return _final(z, r[0], r[1], inv2d)
